# baseline (speedup 1.0000x reference)
import functools

import jax
import jax.numpy as jnp
from jax import lax
from jax.experimental import pallas as pl
from jax.experimental.pallas import tpu as pltpu
from jax.experimental.pallas import tpu_sc as plsc

NUM_CORES = 2
NUM_SUBCORES = 16
LANES = 16
NW = NUM_CORES * NUM_SUBCORES

BATCH = 16384
FEAT_DIM = 32
BPW = BATCH // NW
IDX_CHUNK = 128
NCHUNK = BPW // IDX_CHUNK


def _partial_sums(features, labels, centers):
    mesh = plsc.VectorSubcoreMesh(core_axis_name="c", subcore_axis_name="s")

    @functools.partial(
        pl.kernel,
        mesh=mesh,
        out_type=jax.ShapeDtypeStruct((NW, LANES), jnp.float32),
        compiler_params=pltpu.CompilerParams(use_tc_tiling_on_sc=False),
        scratch_types=[
            pltpu.VMEM((NCHUNK, IDX_CHUNK), jnp.int32),
            pltpu.VMEM((BPW, FEAT_DIM), jnp.float32),
            pltpu.VMEM((BPW, FEAT_DIM), jnp.float32),
            pltpu.VMEM((LANES,), jnp.float32),
            pltpu.SemaphoreType.DMA,
        ],
    )
    def k(feat_hbm, lab_hbm, cent_hbm, out_hbm, idx_v, feat_v, rows_v, acc_v, sem):
        wid = lax.axis_index("s") * NUM_CORES + lax.axis_index("c")
        base = wid * BPW

        for j in range(NCHUNK):
            pltpu.sync_copy(lab_hbm.at[pl.ds(base + j * IDX_CHUNK, IDX_CHUNK)],
                            idx_v.at[j])

        copies = [
            pltpu.async_copy(cent_hbm.at[idx_v.at[j]],
                             rows_v.at[pl.ds(j * IDX_CHUNK, IDX_CHUNK)],
                             sem)
            for j in range(NCHUNK)
        ]
        pltpu.sync_copy(feat_hbm.at[pl.ds(base, BPW), :], feat_v)
        for c in copies:
            c.wait()

        def body(i, acc):
            a0, a1 = acc
            d0 = feat_v[i, pl.ds(0, LANES)] - rows_v[i, pl.ds(0, LANES)]
            d1 = feat_v[i, pl.ds(LANES, LANES)] - rows_v[i, pl.ds(LANES, LANES)]
            return (a0 + d0 * d0, a1 + d1 * d1)

        zero = jnp.zeros((LANES,), jnp.float32)
        a0, a1 = lax.fori_loop(0, BPW, body, (zero, zero))
        acc_v[...] = a0 + a1
        pltpu.sync_copy(acc_v, out_hbm.at[wid])

    return k(features, labels, centers)


def kernel(features, labels, centers):
    partials = _partial_sums(features, labels.astype(jnp.int32), centers)
    return jnp.sum(partials) * (0.5 / BATCH)


# speedup vs baseline: 4.3925x; 4.3925x over previous
import functools

import jax
import jax.numpy as jnp
from jax import lax
from jax.experimental import pallas as pl
from jax.experimental.pallas import tpu as pltpu
from jax.experimental.pallas import tpu_sc as plsc

NUM_CORES = 2
NUM_SUBCORES = 16
LANES = 16
NW = NUM_CORES * NUM_SUBCORES

BATCH = 16384
FEAT_DIM = 32
BPW = BATCH // NW
GRP = BPW // LANES
TILE_W = 128
NBUF = 16


def _partial_sums(features_t, labels, centers_t):
    mesh = plsc.VectorSubcoreMesh(core_axis_name="c", subcore_axis_name="s")

    @functools.partial(
        pl.kernel,
        mesh=mesh,
        out_type=jax.ShapeDtypeStruct((NW * LANES,), jnp.float32),
        compiler_params=pltpu.CompilerParams(use_tc_tiling_on_sc=True,
                                             needs_layout_passes=False),
        scratch_types=[
            pltpu.VMEM((BPW,), jnp.int32),
            pltpu.VMEM((FEAT_DIM, BPW), jnp.float32),
            pltpu.VMEM((NBUF, FEAT_DIM, TILE_W), jnp.float32),
            pltpu.VMEM((LANES,), jnp.float32),
            pltpu.SemaphoreType.DMA,
            pltpu.SemaphoreType.DMA,
        ],
    )
    def k(feat_hbm, lab_hbm, cent_hbm, out_hbm,
          idx_v, feat_v, cbuf, acc_v, sem, fsem):
        wid = lax.axis_index("s") * NUM_CORES + lax.axis_index("c")
        base = wid * BPW

        pltpu.sync_copy(lab_hbm.at[pl.ds(base, BPW)], idx_v)
        feat_cp = pltpu.async_copy(feat_hbm.at[:, pl.ds(base, BPW)],
                                   feat_v, fsem)

        iota = lax.iota(jnp.int32, LANES)
        rows_lo = iota
        rows_hi = iota + LANES

        def fire(vec, i):
            col = pl.multiple_of((vec[i] >> 7) << 7, TILE_W)
            pltpu.async_copy(cent_hbm.at[:, pl.ds(col, TILE_W)],
                             cbuf.at[i], sem)

        def consume(vec, i, g, acc):
            pltpu.make_async_copy(cent_hbm.at[:, pl.ds(0, TILE_W)],
                                  cbuf.at[i], sem).wait()
            lane = jnp.full((LANES,), vec[i] & 127, jnp.int32)
            slot = jnp.full((LANES,), i, jnp.int32)
            item = jnp.full((LANES,), g * LANES + i, jnp.int32)
            c_lo = plsc.load_gather(cbuf, [slot, rows_lo, lane])
            c_hi = plsc.load_gather(cbuf, [slot, rows_hi, lane])
            f_lo = plsc.load_gather(feat_v, [rows_lo, item])
            f_hi = plsc.load_gather(feat_v, [rows_hi, item])
            d_lo = f_lo - c_lo
            d_hi = f_hi - c_hi
            return acc + d_lo * d_lo + d_hi * d_hi

        vec0 = idx_v[pl.ds(0, LANES)]
        for i in range(LANES):
            fire(vec0, i)
        feat_cp.wait()

        def body(g, carry):
            vec, acc = carry
            vec_next = idx_v[pl.ds((g + 1) * LANES, LANES)]
            for i in range(LANES):
                acc = consume(vec, i, g, acc)
                fire(vec_next, i)
            return vec_next, acc

        vec_last, acc = lax.fori_loop(
            0, GRP - 1, body, (vec0, jnp.zeros((LANES,), jnp.float32)))

        for i in range(LANES):
            acc = consume(vec_last, i, GRP - 1, acc)

        acc_v[...] = acc
        pltpu.sync_copy(acc_v, out_hbm.at[pl.ds(wid * LANES, LANES)])

    return k(features_t, labels, centers_t)


def kernel(features, labels, centers):
    partials = _partial_sums(features.T, labels.astype(jnp.int32), centers.T)
    return jnp.sum(partials) * (0.5 / BATCH)
